# baseline (speedup 1.0000x reference)
import functools

import jax
import jax.numpy as jnp
from jax import lax
from jax.experimental import pallas as pl
from jax.experimental.pallas import tpu as pltpu
from jax.experimental.pallas import tpu_sc as plsc

C = 128
N0, N1, N2 = 10000, 40000, 20000
NC, NS = 2, 16
NW = NC * NS
KG = 128
KFIRE = 5
KGS = 128
EALIGN = 12288
BE = 768
BIG = 1 << 30


def _ceil_to(x, m):
    return -(-x // m) * m


def _mm_body(x_ref, w_ref, b_ref, o_ref):
    o_ref[...] = jnp.dot(x_ref[...], w_ref[...],
                         preferred_element_type=jnp.float32) + b_ref[...]


def _mm(x, w, b, bn):
    n, k = x.shape
    m = w.shape[1]
    return pl.pallas_call(
        _mm_body,
        grid=(n // bn,),
        in_specs=[pl.BlockSpec((bn, k), lambda i: (i, 0)),
                  pl.BlockSpec((k, m), lambda i: (0, 0)),
                  pl.BlockSpec((1, m), lambda i: (0, 0))],
        out_specs=pl.BlockSpec((bn, m), lambda i: (i, 0)),
        out_shape=jax.ShapeDtypeStruct((n, m), jnp.float32),
        compiler_params=pltpu.CompilerParams(
            dimension_semantics=("parallel",)),
    )(x, w, b.reshape(1, m))


def _edge_body(kinv, ga_ref, gb_ref, invT_ref, w1c_ref, w2_ref, b2_ref,
               winfT_ref, binf_ref, o_ref):
    g = ga_ref[...] + gb_ref[...]
    invT = invT_ref[...]
    w1c = w1c_ref[...]
    for k in range(kinv):
        g += invT[k, :][:, None] * w1c[k, :][None, :]
    h = jax.nn.silu(g)
    h2 = jax.nn.silu(jnp.dot(h, w2_ref[...],
                             preferred_element_type=jnp.float32) + b2_ref[...])
    gate = jax.nn.sigmoid(
        jnp.sum(h2 * winfT_ref[...], axis=1, keepdims=True) + binf_ref[0, 0])
    o_ref[...] = h2 * gate


def _edge_stage(gout, offa, offb, ep, invT, p, kinv, be=BE):
    w1c = jnp.zeros((8, C), jnp.float32).at[:kinv].set(p['W1'][2 * C:2 * C + kinv])
    body = functools.partial(_edge_body, kinv)
    ba, bb = offa // be, offb // be
    return pl.pallas_call(
        body,
        grid=(ep // be,),
        in_specs=[pl.BlockSpec((be, C), lambda i, ba=ba: (ba + i, 0)),
                  pl.BlockSpec((be, C), lambda i, bb=bb: (bb + i, 0)),
                  pl.BlockSpec((8, be), lambda i: (0, i)),
                  pl.BlockSpec((8, C), lambda i: (0, 0)),
                  pl.BlockSpec((C, C), lambda i: (0, 0)),
                  pl.BlockSpec((1, C), lambda i: (0, 0)),
                  pl.BlockSpec((1, C), lambda i: (0, 0)),
                  pl.BlockSpec((1, 1), lambda i: (0, 0))],
        out_specs=pl.BlockSpec((be, C), lambda i: (i, 0)),
        out_shape=jax.ShapeDtypeStruct((ep, C), jnp.float32),
        compiler_params=pltpu.CompilerParams(
            dimension_semantics=("parallel",)),
    )(gout, gout, invT, w1c, p['W2'], p['b2'].reshape(1, C),
      p['Winf'].reshape(1, C), p['binf'].reshape(1, 1))


def _upd_body(p_ref, x_ref, u1_ref, ub1_ref, u2_ref, ub2_ref, o_ref):
    agg = p_ref[0] + p_ref[1]
    a = jax.nn.silu(agg)
    t = jax.nn.silu(jnp.dot(a, u1_ref[...],
                            preferred_element_type=jnp.float32) + ub1_ref[...])
    o_ref[...] = x_ref[...] + jnp.dot(
        t, u2_ref[...], preferred_element_type=jnp.float32) + ub2_ref[...]


def _update(parts, x, u, bn=1000):
    n = x.shape[0]
    return pl.pallas_call(
        _upd_body,
        grid=(n // bn,),
        in_specs=[pl.BlockSpec((2, bn, C), lambda i: (0, i, 0)),
                  pl.BlockSpec((bn, C), lambda i: (i, 0)),
                  pl.BlockSpec((C, C), lambda i: (0, 0)),
                  pl.BlockSpec((1, C), lambda i: (0, 0)),
                  pl.BlockSpec((C, C), lambda i: (0, 0)),
                  pl.BlockSpec((1, C), lambda i: (0, 0))],
        out_specs=pl.BlockSpec((bn, C), lambda i: (i, 0)),
        out_shape=jax.ShapeDtypeStruct((n, C), jnp.float32),
        compiler_params=pltpu.CompilerParams(
            dimension_semantics=("parallel",)),
    )(parts, x, u['U1'], u['ub1'].reshape(1, C),
      u['U2'], u['ub2'].reshape(1, C))


def _sc_gather(table, idx):
    etot = int(idx.shape[0])
    assert etot % (NW * KG * KFIRE) == 0
    lw = etot // NW
    ng = lw // (KG * KFIRE)
    mesh = plsc.VectorSubcoreMesh(core_axis_name="c", subcore_axis_name="s")

    @functools.partial(
        pl.kernel, mesh=mesh,
        out_type=jax.ShapeDtypeStruct((etot, C), jnp.float32),
        scratch_types=[pltpu.VMEM((KFIRE, KG), jnp.int32),
                       pltpu.VMEM((KFIRE * KG, C), jnp.float32),
                       pltpu.SemaphoreType.DMA((KFIRE,)),
                       pltpu.SemaphoreType.DMA((KFIRE,)),
                       pltpu.SemaphoreType.DMA((KFIRE,))])
    def gather_kernel(t_hbm, i_hbm, o_hbm, idx_v, rows_v, sem_i, sem_g, sem_s):
        wid = lax.axis_index("s") * NC + lax.axis_index("c")
        base_w = wid * lw

        @pl.loop(0, ng)
        def _(g):
            start0 = base_w + g * (KFIRE * KG)
            for b in range(KFIRE):
                pltpu.async_copy(i_hbm.at[0, pl.ds(start0 + b * KG, KG)],
                                 idx_v.at[b], sem_i.at[b])
            for b in range(KFIRE):
                pltpu.make_async_copy(i_hbm.at[0, pl.ds(0, KG)],
                                      idx_v.at[b], sem_i.at[b]).wait()

                @pl.when(g > 0)
                def _(b=b):
                    pltpu.make_async_copy(rows_v.at[pl.ds(0, KG)],
                                          o_hbm.at[pl.ds(0, KG)],
                                          sem_s.at[b]).wait()

                pltpu.async_copy(t_hbm.at[idx_v.at[b]],
                                 rows_v.at[pl.ds(b * KG, KG)], sem_g.at[b])
            for b in range(KFIRE):
                pltpu.make_async_copy(t_hbm.at[idx_v.at[b]],
                                      rows_v.at[pl.ds(b * KG, KG)],
                                      sem_g.at[b]).wait()
                pltpu.async_copy(rows_v.at[pl.ds(b * KG, KG)],
                                 o_hbm.at[pl.ds(start0 + b * KG, KG)],
                                 sem_s.at[b])

        for b in range(KFIRE):
            pltpu.make_async_copy(rows_v.at[pl.ds(0, KG)],
                                  o_hbm.at[pl.ds(0, KG)], sem_s.at[b]).wait()

    return gather_kernel(table, idx.reshape(1, etot))


def _sc_scatter(rank_jobs, npads, chs, accs):
    nr = len(rank_jobs)
    accmax = max(accs)
    mesh = plsc.VectorSubcoreMesh(core_axis_name="c", subcore_axis_name="s")
    out_type = [jax.ShapeDtypeStruct((NC, npads[r], C), jnp.float32)
                for r in range(nr)]
    flat_in = []
    for jobs in rank_jobs:
        for m, rv in jobs:
            flat_in.extend((m, rv))
    nin = len(flat_in)

    @functools.partial(
        pl.kernel, mesh=mesh, out_type=out_type,
        scratch_types=[pltpu.VMEM((2 * KGS, C), jnp.float32),
                       pltpu.VMEM((2, KGS), jnp.int32),
                       pltpu.VMEM((KGS,), jnp.int32),
                       pltpu.VMEM((64, C), jnp.float32),
                       pltpu.VMEM_SHARED((accmax, C), jnp.float32),
                       pltpu.SemaphoreType.DMA((2,)),
                       pltpu.SemaphoreType.DMA((2,))])
    def scatter_kernel(*refs):
        in_refs = refs[:nin]
        p_refs = refs[nin:nin + nr]
        mbuf, rbuf, sidx, zeros_v, acc, sem_m, sem_r = refs[nin + nr:]
        cid = lax.axis_index("c")
        sid = lax.axis_index("s")
        wid = sid * NC + cid

        @pl.loop(0, 64)
        def _(r):
            @pl.loop(0, C, step=16)
            def _(cc):
                zeros_v[r, pl.ds(cc, 16)] = jnp.zeros((16,), jnp.float32)

        k = 0
        for r in range(nr):
            jobs = rank_jobs[r]
            job_refs = [(in_refs[k + 2 * t], in_refs[k + 2 * t + 1])
                        for t in range(len(jobs))]
            k += 2 * len(jobs)
            ch, accr = chs[r], accs[r]
            nch = npads[r] // ch
            srows = accr // NS
            erows = ch // NS
            for c in range(nch):
                base_node = c * ch

                @pl.loop(0, srows, step=64)
                def _(r0, srows=srows):
                    pltpu.sync_copy(zeros_v, acc.at[pl.ds(sid * srows + r0, 64)])
                plsc.subcore_barrier()

                for t, (m_ref, r_ref) in enumerate(job_refs):
                    epad = int(rank_jobs[r][t][0].shape[0])
                    nbw = epad // (NW * KGS)

                    def issue(i, b, m_ref=m_ref, r_ref=r_ref, nbw=nbw):
                        base = (wid * nbw + i) * KGS
                        pltpu.async_copy(m_ref.at[pl.ds(base, KGS)],
                                         mbuf.at[pl.ds(b * KGS, KGS)],
                                         sem_m.at[b])
                        pltpu.async_copy(r_ref.at[0, pl.ds(base, KGS)],
                                         rbuf.at[b], sem_r.at[b])

                    issue(0, 0)

                    @pl.loop(0, nbw)
                    def _(i, m_ref=m_ref, r_ref=r_ref, nbw=nbw,
                          base_node=base_node, ch=ch, issue=issue):
                        b = lax.rem(i, 2)
                        nb = 1 - b

                        @pl.when(i + 1 < nbw)
                        def _():
                            issue(i + 1, nb)

                        pltpu.make_async_copy(
                            m_ref.at[pl.ds(0, KGS)],
                            mbuf.at[pl.ds(b * KGS, KGS)], sem_m.at[b]).wait()
                        pltpu.make_async_copy(
                            r_ref.at[0, pl.ds(0, KGS)],
                            rbuf.at[b], sem_r.at[b]).wait()

                        @pl.loop(0, KGS, step=16)
                        def _(tt):
                            rv = rbuf[b, pl.ds(tt, 16)] - base_node
                            ok = (rv >= 0) & (rv < ch)
                            sidx[pl.ds(tt, 16)] = jnp.where(ok, rv, ch)

                        pltpu.sync_copy(mbuf.at[pl.ds(b * KGS, KGS)],
                                        acc.at[sidx], add=True)
                plsc.subcore_barrier()

                pltpu.sync_copy(
                    acc.at[pl.ds(sid * erows, erows)],
                    p_refs[r].at[cid, pl.ds(base_node + sid * erows, erows)])
                plsc.subcore_barrier()

    return scatter_kernel(*flat_in)


def _pad1(a, n, val):
    e = a.shape[0]
    if e == n:
        return a
    return jnp.concatenate(
        [a, jnp.full((n - e,) + a.shape[1:], val, a.dtype)], axis=0)


def kernel(x_0, x_1, x_2, adj_0, adj_1, inc_1, inc_2,
           inv_rr_0, inv_rr_1, inv_rrm1_1, inv_rrm1_2, params):
    p = params
    zb = jnp.zeros((C,), jnp.float32)

    a1 = _mm(x_0, p['same_0']['W1'][:C], zb, 1000)
    b1t = _mm(x_0, p['same_0']['W1'][C:2 * C], p['same_0']['b1'], 1000)
    a3 = _mm(x_0, p['l2h_1']['W1'][:C], zb, 1000)
    a2 = _mm(x_1, p['same_1']['W1'][:C], zb, 1000)
    b2t = _mm(x_1, p['same_1']['W1'][C:2 * C], p['same_1']['b1'], 1000)
    b3 = _mm(x_1, p['l2h_1']['W1'][C:2 * C], p['l2h_1']['b1'], 1000)
    a4 = _mm(x_1, p['l2h_2']['W1'][:C], zb, 1000)
    b4 = _mm(x_2, p['l2h_2']['W1'][C:2 * C], p['l2h_2']['b1'], 1000)

    convs = [
        dict(send=adj_0[0], recv=adj_0[1], inv=inv_rr_0, kinv=3,
             ta=a1, tb=b1t, p=p['same_0'], rank=0),
        dict(send=adj_1[0], recv=adj_1[1], inv=inv_rr_1, kinv=6,
             ta=a2, tb=b2t, p=p['same_1'], rank=1),
        dict(send=inc_1[0], recv=inc_1[1], inv=inv_rrm1_1, kinv=3,
             ta=a3, tb=b3, p=p['l2h_1'], rank=1),
        dict(send=inc_2[0], recv=inc_2[1], inv=inv_rrm1_2, kinv=6,
             ta=a4, tb=b4, p=p['l2h_2'], rank=2),
    ]

    tables = []
    toff = 0
    for cv in convs:
        for key in ('ta', 'tb'):
            cv[key + '_off'] = toff
            tables.append(cv[key])
            toff += cv[key].shape[0]
    table_all = jnp.concatenate(tables, axis=0)

    idx_segs = []
    goff = 0
    for cv in convs:
        e = cv['send'].shape[0]
        ep = _ceil_to(e, EALIGN)
        cv['epad'] = ep
        idx_segs.append(_pad1(cv['send'] + cv['ta_off'], ep, 0))
        idx_segs.append(_pad1(cv['recv'] + cv['tb_off'], ep, 0))
        cv['ga_off'], cv['gb_off'] = goff, goff + ep
        goff += 2 * ep
        cv['recv_s'] = _pad1(cv['recv'], ep, BIG).reshape(1, ep)
        invT = jnp.transpose(cv['inv'])
        invT = jnp.concatenate(
            [invT, jnp.zeros((8 - cv['kinv'], e), jnp.float32)], axis=0)
        cv['invT'] = _pad1(invT.T, ep, 0.0).T
    idx_all = jnp.concatenate(idx_segs, axis=0)
    gtot = _ceil_to(idx_all.shape[0], NW * KG * KFIRE)
    idx_all = _pad1(idx_all, gtot, 0)

    gout = _sc_gather(table_all, idx_all)

    for cv in convs:
        cv['m'] = _edge_stage(gout, cv['ga_off'], cv['gb_off'], cv['epad'],
                              cv['invT'], cv['p'], cv['kinv'])

    sizes = [N0, N1, N2]
    chs, accs, npads, rank_jobs = [], [], [], []
    for r in range(3):
        jobs = [(cv['m'], cv['recv_s']) for cv in convs if cv['rank'] == r]
        nch = max(1, -(-sizes[r] * 4 * C // (5 << 20)))
        ch = _ceil_to(-(-sizes[r] // nch), NS * 8)
        acc = _ceil_to(ch + 1, 1024)
        chs.append(ch)
        accs.append(acc)
        npads.append(nch * ch)
        rank_jobs.append(jobs)
    parts = _sc_scatter(rank_jobs, npads, chs, accs)

    feats = [x_0, x_1, x_2]
    outs = []
    for r in range(3):
        u = p['upd_%d' % r]
        outs.append(_update(parts[r], feats[r], u, 1000))
    return tuple(outs)


# speedup vs baseline: 1.4525x; 1.4525x over previous
import functools

import jax
import jax.numpy as jnp
from jax import lax
from jax.experimental import pallas as pl
from jax.experimental.pallas import tpu as pltpu
from jax.experimental.pallas import tpu_sc as plsc

C = 128
N0, N1, N2 = 10000, 40000, 20000
NC, NS = 2, 16
NW = NC * NS
KG = 256
KFIRE = 2
KGS = 128
EALIGN = 16384
BE = 1024
BIG = 1 << 30


def _ceil_to(x, m):
    return -(-x // m) * m


def _mm_body(x_ref, w_ref, b_ref, o_ref):
    o_ref[...] = jnp.dot(x_ref[...], w_ref[...],
                         preferred_element_type=jnp.float32) + b_ref[...]


def _mm(x, w, b, bn):
    n, k = x.shape
    m = w.shape[1]
    return pl.pallas_call(
        _mm_body,
        grid=(n // bn,),
        in_specs=[pl.BlockSpec((bn, k), lambda i: (i, 0)),
                  pl.BlockSpec((k, m), lambda i: (0, 0)),
                  pl.BlockSpec((1, m), lambda i: (0, 0))],
        out_specs=pl.BlockSpec((bn, m), lambda i: (i, 0)),
        out_shape=jax.ShapeDtypeStruct((n, m), jnp.float32),
        compiler_params=pltpu.CompilerParams(
            dimension_semantics=("parallel",)),
    )(x, w, b.reshape(1, m))


def _edge_body(kinv, ga_ref, gb_ref, invT_ref, w1c_ref, w2_ref, b2_ref,
               winfT_ref, binf_ref, o_ref):
    g = ga_ref[...] + gb_ref[...]
    invT = invT_ref[...]
    w1c = w1c_ref[...]
    for k in range(kinv):
        g += invT[k, :][:, None] * w1c[k, :][None, :]
    h = jax.nn.silu(g)
    h2 = jax.nn.silu(jnp.dot(h, w2_ref[...],
                             preferred_element_type=jnp.float32) + b2_ref[...])
    gate = jax.nn.sigmoid(
        jnp.sum(h2 * winfT_ref[...], axis=1, keepdims=True) + binf_ref[0, 0])
    o_ref[...] = h2 * gate


def _edge_stage(ga, gb, invT, p, kinv, be=BE):
    ep = ga.shape[0]
    w1c = jnp.zeros((8, C), jnp.float32).at[:kinv].set(p['W1'][2 * C:2 * C + kinv])
    body = functools.partial(_edge_body, kinv)
    return pl.pallas_call(
        body,
        grid=(ep // be,),
        in_specs=[pl.BlockSpec((be, C), lambda i: (i, 0)),
                  pl.BlockSpec((be, C), lambda i: (i, 0)),
                  pl.BlockSpec((8, be), lambda i: (0, i)),
                  pl.BlockSpec((8, C), lambda i: (0, 0)),
                  pl.BlockSpec((C, C), lambda i: (0, 0)),
                  pl.BlockSpec((1, C), lambda i: (0, 0)),
                  pl.BlockSpec((1, C), lambda i: (0, 0)),
                  pl.BlockSpec((1, 1), lambda i: (0, 0))],
        out_specs=pl.BlockSpec((be, C), lambda i: (i, 0)),
        out_shape=jax.ShapeDtypeStruct((ep, C), jnp.float32),
        compiler_params=pltpu.CompilerParams(
            dimension_semantics=("parallel",)),
    )(ga, gb, invT, w1c, p['W2'], p['b2'].reshape(1, C),
      p['Winf'].reshape(1, C), p['binf'].reshape(1, 1))


def _upd_body(p_ref, x_ref, u1_ref, ub1_ref, u2_ref, ub2_ref, o_ref):
    agg = p_ref[0] + p_ref[1]
    a = jax.nn.silu(agg)
    t = jax.nn.silu(jnp.dot(a, u1_ref[...],
                            preferred_element_type=jnp.float32) + ub1_ref[...])
    o_ref[...] = x_ref[...] + jnp.dot(
        t, u2_ref[...], preferred_element_type=jnp.float32) + ub2_ref[...]


def _update(parts, x, u, bn=1000):
    n = x.shape[0]
    return pl.pallas_call(
        _upd_body,
        grid=(n // bn,),
        in_specs=[pl.BlockSpec((2, bn, C), lambda i: (0, i, 0)),
                  pl.BlockSpec((bn, C), lambda i: (i, 0)),
                  pl.BlockSpec((C, C), lambda i: (0, 0)),
                  pl.BlockSpec((1, C), lambda i: (0, 0)),
                  pl.BlockSpec((C, C), lambda i: (0, 0)),
                  pl.BlockSpec((1, C), lambda i: (0, 0))],
        out_specs=pl.BlockSpec((bn, C), lambda i: (i, 0)),
        out_shape=jax.ShapeDtypeStruct((n, C), jnp.float32),
        compiler_params=pltpu.CompilerParams(
            dimension_semantics=("parallel",)),
    )(parts, x, u['U1'], u['ub1'].reshape(1, C),
      u['U2'], u['ub2'].reshape(1, C))


def _sc_gather(tables, idxs):
    nj = len(tables)
    epads = [int(i.shape[0]) for i in idxs]
    mesh = plsc.VectorSubcoreMesh(core_axis_name="c", subcore_axis_name="s")
    out_type = [jax.ShapeDtypeStruct((ep, C), jnp.float32) for ep in epads]

    @functools.partial(
        pl.kernel, mesh=mesh, out_type=out_type,
        scratch_types=([pltpu.VMEM((KG,), jnp.int32)] * KFIRE
                       + [pltpu.VMEM((KFIRE * KG, C), jnp.float32),
                          pltpu.SemaphoreType.DMA((KFIRE,)),
                          pltpu.SemaphoreType.DMA((KFIRE,)),
                          pltpu.SemaphoreType.DMA((KFIRE,))]))
    def gather_kernel(*refs):
        table_refs = refs[:nj]
        idx_refs = refs[nj:2 * nj]
        out_refs = refs[2 * nj:3 * nj]
        idx_v = refs[3 * nj:3 * nj + KFIRE]
        rows_v, sem_i, sem_g, sem_s = refs[3 * nj + KFIRE:]
        wid = lax.axis_index("s") * NC + lax.axis_index("c")
        for j in range(nj):
            t_hbm, i_hbm, o_hbm = table_refs[j], idx_refs[j], out_refs[j]
            lw = epads[j] // NW
            ng = lw // (KG * KFIRE)
            base_w = wid * lw

            @pl.loop(0, ng)
            def _(g, t_hbm=t_hbm, i_hbm=i_hbm, o_hbm=o_hbm, base_w=base_w,
                  first=(j == 0)):
                start0 = base_w + g * (KFIRE * KG)
                for b in range(KFIRE):
                    pltpu.async_copy(i_hbm.at[0, pl.ds(start0 + b * KG, KG)],
                                     idx_v[b], sem_i.at[b])
                for b in range(KFIRE):
                    pltpu.make_async_copy(i_hbm.at[0, pl.ds(0, KG)],
                                          idx_v[b], sem_i.at[b]).wait()

                    if first:
                        @pl.when(g > 0)
                        def _(b=b, o_hbm=o_hbm):
                            pltpu.make_async_copy(
                                rows_v.at[pl.ds(0, KG)],
                                o_hbm.at[pl.ds(0, KG)], sem_s.at[b]).wait()
                    else:
                        pltpu.make_async_copy(
                            rows_v.at[pl.ds(0, KG)],
                            o_hbm.at[pl.ds(0, KG)], sem_s.at[b]).wait()

                    pltpu.async_copy(t_hbm.at[idx_v[b]],
                                     rows_v.at[pl.ds(b * KG, KG)], sem_g.at[b])
                for b in range(KFIRE):
                    pltpu.make_async_copy(t_hbm.at[idx_v[b]],
                                          rows_v.at[pl.ds(b * KG, KG)],
                                          sem_g.at[b]).wait()
                    pltpu.async_copy(rows_v.at[pl.ds(b * KG, KG)],
                                     o_hbm.at[pl.ds(start0 + b * KG, KG)],
                                     sem_s.at[b])

        for b in range(KFIRE):
            pltpu.make_async_copy(rows_v.at[pl.ds(0, KG)],
                                  out_refs[-1].at[pl.ds(0, KG)],
                                  sem_s.at[b]).wait()

    return gather_kernel(*tables, *[i.reshape(1, -1) for i in idxs])


def _sc_scatter(rank_jobs, npads, chs, accs):
    nr = len(rank_jobs)
    accmax = max(accs)
    mesh = plsc.VectorSubcoreMesh(core_axis_name="c", subcore_axis_name="s")
    out_type = [jax.ShapeDtypeStruct((NC, npads[r], C), jnp.float32)
                for r in range(nr)]
    flat_in = []
    for jobs in rank_jobs:
        for m, rv in jobs:
            flat_in.extend((m, rv))
    nin = len(flat_in)

    @functools.partial(
        pl.kernel, mesh=mesh, out_type=out_type,
        scratch_types=[pltpu.VMEM((2 * KGS, C), jnp.float32),
                       pltpu.VMEM((2, KGS), jnp.int32),
                       pltpu.VMEM((KGS,), jnp.int32),
                       pltpu.VMEM((64, C), jnp.float32),
                       pltpu.VMEM_SHARED((accmax, C), jnp.float32),
                       pltpu.SemaphoreType.DMA((2,)),
                       pltpu.SemaphoreType.DMA((2,))])
    def scatter_kernel(*refs):
        in_refs = refs[:nin]
        p_refs = refs[nin:nin + nr]
        mbuf, rbuf, sidx, zeros_v, acc, sem_m, sem_r = refs[nin + nr:]
        cid = lax.axis_index("c")
        sid = lax.axis_index("s")
        wid = sid * NC + cid

        @pl.loop(0, 64)
        def _(r):
            @pl.loop(0, C, step=16)
            def _(cc):
                zeros_v[r, pl.ds(cc, 16)] = jnp.zeros((16,), jnp.float32)

        k = 0
        for r in range(nr):
            jobs = rank_jobs[r]
            job_refs = [(in_refs[k + 2 * t], in_refs[k + 2 * t + 1])
                        for t in range(len(jobs))]
            k += 2 * len(jobs)
            ch, accr = chs[r], accs[r]
            nch = npads[r] // ch
            srows = accr // NS
            erows = ch // NS
            for c in range(nch):
                base_node = c * ch

                @pl.loop(0, srows, step=64)
                def _(r0, srows=srows):
                    pltpu.sync_copy(zeros_v, acc.at[pl.ds(sid * srows + r0, 64)])
                plsc.subcore_barrier()

                for t, (m_ref, r_ref) in enumerate(job_refs):
                    epad = int(rank_jobs[r][t][0].shape[0])
                    nbw = epad // (NW * KGS)

                    def issue(i, b, m_ref=m_ref, r_ref=r_ref, nbw=nbw):
                        base = (wid * nbw + i) * KGS
                        pltpu.async_copy(m_ref.at[pl.ds(base, KGS)],
                                         mbuf.at[pl.ds(b * KGS, KGS)],
                                         sem_m.at[b])
                        pltpu.async_copy(r_ref.at[0, pl.ds(base, KGS)],
                                         rbuf.at[b], sem_r.at[b])

                    issue(0, 0)

                    @pl.loop(0, nbw)
                    def _(i, m_ref=m_ref, r_ref=r_ref, nbw=nbw,
                          base_node=base_node, ch=ch, issue=issue):
                        b = lax.rem(i, 2)
                        nb = 1 - b

                        @pl.when(i + 1 < nbw)
                        def _():
                            issue(i + 1, nb)

                        pltpu.make_async_copy(
                            m_ref.at[pl.ds(0, KGS)],
                            mbuf.at[pl.ds(b * KGS, KGS)], sem_m.at[b]).wait()
                        pltpu.make_async_copy(
                            r_ref.at[0, pl.ds(0, KGS)],
                            rbuf.at[b], sem_r.at[b]).wait()

                        @pl.loop(0, KGS, step=16)
                        def _(tt):
                            rv = rbuf[b, pl.ds(tt, 16)] - base_node
                            ok = (rv >= 0) & (rv < ch)
                            sidx[pl.ds(tt, 16)] = jnp.where(ok, rv, ch)

                        pltpu.sync_copy(mbuf.at[pl.ds(b * KGS, KGS)],
                                        acc.at[sidx], add=True)
                plsc.subcore_barrier()

                pltpu.sync_copy(
                    acc.at[pl.ds(sid * erows, erows)],
                    p_refs[r].at[cid, pl.ds(base_node + sid * erows, erows)])
                plsc.subcore_barrier()

    return scatter_kernel(*flat_in)


def _pad1(a, n, val):
    e = a.shape[0]
    if e == n:
        return a
    return jnp.concatenate(
        [a, jnp.full((n - e,) + a.shape[1:], val, a.dtype)], axis=0)


def kernel(x_0, x_1, x_2, adj_0, adj_1, inc_1, inc_2,
           inv_rr_0, inv_rr_1, inv_rrm1_1, inv_rrm1_2, params):
    p = params
    zb = jnp.zeros((C,), jnp.float32)

    a1 = _mm(x_0, p['same_0']['W1'][:C], zb, 1000)
    b1t = _mm(x_0, p['same_0']['W1'][C:2 * C], p['same_0']['b1'], 1000)
    a3 = _mm(x_0, p['l2h_1']['W1'][:C], zb, 1000)
    a2 = _mm(x_1, p['same_1']['W1'][:C], zb, 1000)
    b2t = _mm(x_1, p['same_1']['W1'][C:2 * C], p['same_1']['b1'], 1000)
    b3 = _mm(x_1, p['l2h_1']['W1'][C:2 * C], p['l2h_1']['b1'], 1000)
    a4 = _mm(x_1, p['l2h_2']['W1'][:C], zb, 1000)
    b4 = _mm(x_2, p['l2h_2']['W1'][C:2 * C], p['l2h_2']['b1'], 1000)

    convs = [
        dict(send=adj_0[0], recv=adj_0[1], inv=inv_rr_0, kinv=3,
             ta=a1, tb=b1t, p=p['same_0'], rank=0),
        dict(send=adj_1[0], recv=adj_1[1], inv=inv_rr_1, kinv=6,
             ta=a2, tb=b2t, p=p['same_1'], rank=1),
        dict(send=inc_1[0], recv=inc_1[1], inv=inv_rrm1_1, kinv=3,
             ta=a3, tb=b3, p=p['l2h_1'], rank=1),
        dict(send=inc_2[0], recv=inc_2[1], inv=inv_rrm1_2, kinv=6,
             ta=a4, tb=b4, p=p['l2h_2'], rank=2),
    ]

    gt, gi = [], []
    for cv in convs:
        e = cv['send'].shape[0]
        ep = _ceil_to(e, EALIGN)
        cv['epad'] = ep
        gt.extend((cv['ta'], cv['tb']))
        gi.extend((_pad1(cv['send'], ep, 0), _pad1(cv['recv'], ep, 0)))
        cv['recv_s'] = _pad1(cv['recv'], ep, BIG).reshape(1, ep)
        invT = jnp.transpose(cv['inv'])
        invT = jnp.concatenate(
            [invT, jnp.zeros((8 - cv['kinv'], e), jnp.float32)], axis=0)
        cv['invT'] = _pad1(invT.T, ep, 0.0).T

    gout = _sc_gather(gt, gi)

    for j, cv in enumerate(convs):
        cv['m'] = _edge_stage(gout[2 * j], gout[2 * j + 1],
                              cv['invT'], cv['p'], cv['kinv'])

    sizes = [N0, N1, N2]
    chs, accs, npads, rank_jobs = [], [], [], []
    for r in range(3):
        jobs = [(cv['m'], cv['recv_s']) for cv in convs if cv['rank'] == r]
        nch = max(1, -(-sizes[r] * 4 * C // (5 << 20)))
        ch = _ceil_to(-(-sizes[r] // nch), NS * 8)
        acc = _ceil_to(ch + 1, 1024)
        chs.append(ch)
        accs.append(acc)
        npads.append(nch * ch)
        rank_jobs.append(jobs)
    parts = _sc_scatter(rank_jobs, npads, chs, accs)

    feats = [x_0, x_1, x_2]
    outs = []
    for r in range(3):
        u = p['upd_%d' % r]
        outs.append(_update(parts[r], feats[r], u, 1000))
    return tuple(outs)


# speedup vs baseline: 1.6843x; 1.1596x over previous
import functools

import jax
import jax.numpy as jnp
from jax import lax
from jax.experimental import pallas as pl
from jax.experimental.pallas import tpu as pltpu
from jax.experimental.pallas import tpu_sc as plsc

C = 128
N0, N1, N2 = 10000, 40000, 20000
NC, NS = 2, 16
NW = NC * NS
KG = 256
KFIRE = 2
KGS = 128
EALIGN = 16384
BE = 1024
BIG = 1 << 30


def _ceil_to(x, m):
    return -(-x // m) * m


def _mm_body(x_ref, w_ref, b_ref, o_ref):
    o_ref[...] = jnp.dot(x_ref[...], w_ref[...],
                         preferred_element_type=jnp.float32) + b_ref[...]


def _mm(x, w, b, bn):
    n, k = x.shape
    m = w.shape[1]
    return pl.pallas_call(
        _mm_body,
        grid=(n // bn,),
        in_specs=[pl.BlockSpec((bn, k), lambda i: (i, 0)),
                  pl.BlockSpec((k, m), lambda i: (0, 0)),
                  pl.BlockSpec((1, m), lambda i: (0, 0))],
        out_specs=pl.BlockSpec((bn, m), lambda i: (i, 0)),
        out_shape=jax.ShapeDtypeStruct((n, m), jnp.float32),
        compiler_params=pltpu.CompilerParams(
            dimension_semantics=("parallel",)),
    )(x, w, b.reshape(1, m))


def _edge_body(kinv, ga_ref, gb_ref, invT_ref, w1c_ref, w2_ref, b2_ref,
               winfT_ref, binf_ref, o_ref):
    g = ga_ref[...] + gb_ref[...]
    invT = invT_ref[...]
    w1c = w1c_ref[...]
    for k in range(kinv):
        g += invT[k, :][:, None] * w1c[k, :][None, :]
    h = jax.nn.silu(g)
    h2 = jax.nn.silu(jnp.dot(h, w2_ref[...],
                             preferred_element_type=jnp.float32) + b2_ref[...])
    gate = jax.nn.sigmoid(
        jnp.sum(h2 * winfT_ref[...], axis=1, keepdims=True) + binf_ref[0, 0])
    o_ref[...] = h2 * gate


def _edge_stage(ga, gb, invT, p, kinv, be=BE):
    ep = ga.shape[0]
    w1c = jnp.zeros((8, C), jnp.float32).at[:kinv].set(p['W1'][2 * C:2 * C + kinv])
    body = functools.partial(_edge_body, kinv)
    return pl.pallas_call(
        body,
        grid=(ep // be,),
        in_specs=[pl.BlockSpec((be, C), lambda i: (i, 0)),
                  pl.BlockSpec((be, C), lambda i: (i, 0)),
                  pl.BlockSpec((8, be), lambda i: (0, i)),
                  pl.BlockSpec((8, C), lambda i: (0, 0)),
                  pl.BlockSpec((C, C), lambda i: (0, 0)),
                  pl.BlockSpec((1, C), lambda i: (0, 0)),
                  pl.BlockSpec((1, C), lambda i: (0, 0)),
                  pl.BlockSpec((1, 1), lambda i: (0, 0))],
        out_specs=pl.BlockSpec((be, C), lambda i: (i, 0)),
        out_shape=jax.ShapeDtypeStruct((ep, C), jnp.float32),
        compiler_params=pltpu.CompilerParams(
            dimension_semantics=("parallel",)),
    )(ga, gb, invT, w1c, p['W2'], p['b2'].reshape(1, C),
      p['Winf'].reshape(1, C), p['binf'].reshape(1, 1))


def _upd_body(p_ref, x_ref, u1_ref, ub1_ref, u2_ref, ub2_ref, o_ref):
    agg = p_ref[0] + p_ref[1]
    a = jax.nn.silu(agg)
    t = jax.nn.silu(jnp.dot(a, u1_ref[...],
                            preferred_element_type=jnp.float32) + ub1_ref[...])
    o_ref[...] = x_ref[...] + jnp.dot(
        t, u2_ref[...], preferred_element_type=jnp.float32) + ub2_ref[...]


def _update(parts, x, u, bn=1000):
    n = x.shape[0]
    return pl.pallas_call(
        _upd_body,
        grid=(n // bn,),
        in_specs=[pl.BlockSpec((2, bn, C), lambda i: (0, i, 0)),
                  pl.BlockSpec((bn, C), lambda i: (i, 0)),
                  pl.BlockSpec((C, C), lambda i: (0, 0)),
                  pl.BlockSpec((1, C), lambda i: (0, 0)),
                  pl.BlockSpec((C, C), lambda i: (0, 0)),
                  pl.BlockSpec((1, C), lambda i: (0, 0))],
        out_specs=pl.BlockSpec((bn, C), lambda i: (i, 0)),
        out_shape=jax.ShapeDtypeStruct((n, C), jnp.float32),
        compiler_params=pltpu.CompilerParams(
            dimension_semantics=("parallel",)),
    )(parts, x, u['U1'], u['ub1'].reshape(1, C),
      u['U2'], u['ub2'].reshape(1, C))


def _sc_gather(tables, idxs):
    nj = len(tables)
    epads = [int(i.shape[0]) for i in idxs]
    mesh = plsc.VectorSubcoreMesh(core_axis_name="c", subcore_axis_name="s")
    out_type = [jax.ShapeDtypeStruct((ep, C), jnp.float32) for ep in epads]

    @functools.partial(
        pl.kernel, mesh=mesh, out_type=out_type,
        scratch_types=([pltpu.VMEM((KG,), jnp.int32)] * KFIRE
                       + [pltpu.VMEM((KFIRE * KG, C), jnp.float32),
                          pltpu.SemaphoreType.DMA((KFIRE,)),
                          pltpu.SemaphoreType.DMA((KFIRE,)),
                          pltpu.SemaphoreType.DMA((KFIRE,))]))
    def gather_kernel(*refs):
        table_refs = refs[:nj]
        idx_refs = refs[nj:2 * nj]
        out_refs = refs[2 * nj:3 * nj]
        idx_v = refs[3 * nj:3 * nj + KFIRE]
        rows_v, sem_i, sem_g, sem_s = refs[3 * nj + KFIRE:]
        wid = lax.axis_index("s") * NC + lax.axis_index("c")
        for j in range(nj):
            t_hbm, i_hbm, o_hbm = table_refs[j], idx_refs[j], out_refs[j]
            lw = epads[j] // NW
            ng = lw // (KG * KFIRE)
            base_w = wid * lw

            @pl.loop(0, ng)
            def _(g, t_hbm=t_hbm, i_hbm=i_hbm, o_hbm=o_hbm, base_w=base_w,
                  first=(j == 0)):
                start0 = base_w + g * (KFIRE * KG)
                for b in range(KFIRE):
                    pltpu.async_copy(i_hbm.at[0, pl.ds(start0 + b * KG, KG)],
                                     idx_v[b], sem_i.at[b])
                for b in range(KFIRE):
                    pltpu.make_async_copy(i_hbm.at[0, pl.ds(0, KG)],
                                          idx_v[b], sem_i.at[b]).wait()

                    if first:
                        @pl.when(g > 0)
                        def _(b=b, o_hbm=o_hbm):
                            pltpu.make_async_copy(
                                rows_v.at[pl.ds(0, KG)],
                                o_hbm.at[pl.ds(0, KG)], sem_s.at[b]).wait()
                    else:
                        pltpu.make_async_copy(
                            rows_v.at[pl.ds(0, KG)],
                            o_hbm.at[pl.ds(0, KG)], sem_s.at[b]).wait()

                    pltpu.async_copy(t_hbm.at[idx_v[b]],
                                     rows_v.at[pl.ds(b * KG, KG)], sem_g.at[b])
                for b in range(KFIRE):
                    pltpu.make_async_copy(t_hbm.at[idx_v[b]],
                                          rows_v.at[pl.ds(b * KG, KG)],
                                          sem_g.at[b]).wait()
                    pltpu.async_copy(rows_v.at[pl.ds(b * KG, KG)],
                                     o_hbm.at[pl.ds(start0 + b * KG, KG)],
                                     sem_s.at[b])

        for b in range(KFIRE):
            pltpu.make_async_copy(rows_v.at[pl.ds(0, KG)],
                                  out_refs[-1].at[pl.ds(0, KG)],
                                  sem_s.at[b]).wait()

    return gather_kernel(*tables, *[i.reshape(1, -1) for i in idxs])


def _sc_scatter(rank_jobs, npads, chs, accs):
    nr = len(rank_jobs)
    accmax = max(accs)
    mesh = plsc.VectorSubcoreMesh(core_axis_name="c", subcore_axis_name="s")
    out_type = [jax.ShapeDtypeStruct((NC, npads[r], C), jnp.float32)
                for r in range(nr)]
    flat_in = []
    for jobs in rank_jobs:
        for m, rv in jobs:
            flat_in.extend((m, rv))
    nin = len(flat_in)

    @functools.partial(
        pl.kernel, mesh=mesh, out_type=out_type,
        scratch_types=[pltpu.VMEM((2 * KGS, C), jnp.float32),
                       pltpu.VMEM((2, KGS), jnp.int32),
                       pltpu.VMEM((KGS,), jnp.int32),
                       pltpu.VMEM((64, C), jnp.float32),
                       pltpu.VMEM_SHARED((accmax, C), jnp.float32),
                       pltpu.SemaphoreType.DMA((2,)),
                       pltpu.SemaphoreType.DMA((2,))])
    def scatter_kernel(*refs):
        in_refs = refs[:nin]
        p_refs = refs[nin:nin + nr]
        mbuf, rbuf, sidx, zeros_v, acc, sem_m, sem_r = refs[nin + nr:]
        cid = lax.axis_index("c")
        sid = lax.axis_index("s")
        wid = sid * NC + cid

        @pl.loop(0, 64)
        def _(r):
            @pl.loop(0, C, step=16)
            def _(cc):
                zeros_v[r, pl.ds(cc, 16)] = jnp.zeros((16,), jnp.float32)

        k = 0
        for r in range(nr):
            jobs = rank_jobs[r]
            job_refs = [(in_refs[k + 2 * t], in_refs[k + 2 * t + 1])
                        for t in range(len(jobs))]
            k += 2 * len(jobs)
            ch, accr = chs[r], accs[r]
            nch = npads[r] // ch
            srows = accr // NS
            erows = ch // NS
            for c in range(nch):
                base_node = c * ch

                @pl.loop(0, srows, step=64)
                def _(r0, srows=srows):
                    pltpu.sync_copy(zeros_v, acc.at[pl.ds(sid * srows + r0, 64)])
                plsc.subcore_barrier()

                for t, (m_ref, r_ref) in enumerate(job_refs):
                    epad = int(rank_jobs[r][t][0].shape[0])
                    nbw = epad // (NW * KGS)

                    def issue(i, b, m_ref=m_ref, r_ref=r_ref, nbw=nbw):
                        base = (wid * nbw + i) * KGS
                        pltpu.async_copy(m_ref.at[pl.ds(base, KGS)],
                                         mbuf.at[pl.ds(b * KGS, KGS)],
                                         sem_m.at[b])
                        pltpu.async_copy(r_ref.at[0, pl.ds(base, KGS)],
                                         rbuf.at[b], sem_r.at[b])

                    issue(0, 0)

                    @pl.loop(0, nbw)
                    def _(i, m_ref=m_ref, r_ref=r_ref, nbw=nbw,
                          base_node=base_node, ch=ch, issue=issue):
                        b = lax.rem(i, 2)
                        nb = 1 - b

                        @pl.when(i + 1 < nbw)
                        def _():
                            issue(i + 1, nb)

                        pltpu.make_async_copy(
                            m_ref.at[pl.ds(0, KGS)],
                            mbuf.at[pl.ds(b * KGS, KGS)], sem_m.at[b]).wait()
                        pltpu.make_async_copy(
                            r_ref.at[0, pl.ds(0, KGS)],
                            rbuf.at[b], sem_r.at[b]).wait()

                        @pl.loop(0, KGS, step=16)
                        def _(tt):
                            rv = rbuf[b, pl.ds(tt, 16)] - base_node
                            ok = (rv >= 0) & (rv < ch)
                            sidx[pl.ds(tt, 16)] = jnp.where(ok, rv, ch)

                        pltpu.sync_copy(mbuf.at[pl.ds(b * KGS, KGS)],
                                        acc.at[sidx], add=True)
                plsc.subcore_barrier()

                pltpu.sync_copy(
                    acc.at[pl.ds(sid * erows, erows)],
                    p_refs[r].at[cid, pl.ds(base_node + sid * erows, erows)])
                plsc.subcore_barrier()

    return scatter_kernel(*flat_in)


def _pad1(a, n, val):
    e = a.shape[0]
    if e == n:
        return a
    return jnp.concatenate(
        [a, jnp.full((n - e,) + a.shape[1:], val, a.dtype)], axis=0)


def kernel(x_0, x_1, x_2, adj_0, adj_1, inc_1, inc_2,
           inv_rr_0, inv_rr_1, inv_rrm1_1, inv_rrm1_2, params):
    p = params
    zb = jnp.zeros((C,), jnp.float32)

    a1 = _mm(x_0, p['same_0']['W1'][:C], zb, 1000)
    b1t = _mm(x_0, p['same_0']['W1'][C:2 * C], p['same_0']['b1'], 1000)
    a3 = _mm(x_0, p['l2h_1']['W1'][:C], zb, 1000)
    a2 = _mm(x_1, p['same_1']['W1'][:C], zb, 1000)
    b2t = _mm(x_1, p['same_1']['W1'][C:2 * C], p['same_1']['b1'], 1000)
    b3 = _mm(x_1, p['l2h_1']['W1'][C:2 * C], p['l2h_1']['b1'], 1000)
    a4 = _mm(x_1, p['l2h_2']['W1'][:C], zb, 1000)
    b4 = _mm(x_2, p['l2h_2']['W1'][C:2 * C], p['l2h_2']['b1'], 1000)

    convs = [
        dict(send=adj_0[0], recv=adj_0[1], inv=inv_rr_0, kinv=3,
             ta=a1, tb=b1t, p=p['same_0'], rank=0),
        dict(send=adj_1[0], recv=adj_1[1], inv=inv_rr_1, kinv=6,
             ta=a2, tb=b2t, p=p['same_1'], rank=1),
        dict(send=inc_1[0], recv=inc_1[1], inv=inv_rrm1_1, kinv=3,
             ta=a3, tb=b3, p=p['l2h_1'], rank=1),
        dict(send=inc_2[0], recv=inc_2[1], inv=inv_rrm1_2, kinv=6,
             ta=a4, tb=b4, p=p['l2h_2'], rank=2),
    ]

    gt, gi = [], []
    for cv in convs:
        e = cv['send'].shape[0]
        ep = _ceil_to(e, EALIGN)
        cv['epad'] = ep
        gt.extend((cv['ta'], cv['tb']))
        gi.extend((_pad1(cv['send'], ep, 0), _pad1(cv['recv'], ep, 0)))
        cv['recv_s'] = _pad1(cv['recv'], ep, BIG).reshape(1, ep)
        invT = jnp.transpose(cv['inv'])
        invT = jnp.concatenate(
            [invT, jnp.zeros((8 - cv['kinv'], e), jnp.float32)], axis=0)
        cv['invT'] = _pad1(invT.T, ep, 0.0).T

    for j, cv in enumerate(convs):
        ga, gb = _sc_gather([cv['ta'], cv['tb']],
                            [gi[2 * j], gi[2 * j + 1]])
        cv['ga'], cv['gb'] = ga, gb

    for cv in convs:
        cv['m'] = _edge_stage(cv['ga'], cv['gb'],
                              cv['invT'], cv['p'], cv['kinv'])

    sizes = [N0, N1, N2]
    feats = [x_0, x_1, x_2]
    outs = []
    for r in range(3):
        jobs = [(cv['m'], cv['recv_s']) for cv in convs if cv['rank'] == r]
        nch = max(1, -(-sizes[r] * 4 * C // (5 << 20)))
        ch = _ceil_to(-(-sizes[r] // nch), NS * 8)
        acc = _ceil_to(ch + 1, 1024)
        parts = _sc_scatter([jobs], [nch * ch], [ch], [acc])[0]
        u = p['upd_%d' % r]
        outs.append(_update(parts, feats[r], u, 1000))
    return tuple(outs)


# speedup vs baseline: 1.7503x; 1.0392x over previous
import functools

import jax
import jax.numpy as jnp
from jax import lax
from jax.experimental import pallas as pl
from jax.experimental.pallas import tpu as pltpu
from jax.experimental.pallas import tpu_sc as plsc

C = 128
N0, N1, N2 = 10000, 40000, 20000
NC, NS = 2, 16
NW = NC * NS
KG = 256
KFIRE = 2
KGS = 128
EALIGN = 16384
BE = 1024
BIG = 1 << 30


def _ceil_to(x, m):
    return -(-x // m) * m


def _mm_body(x_ref, w_ref, b_ref, o_ref):
    o_ref[...] = (jnp.dot(x_ref[...], w_ref[...],
                          preferred_element_type=jnp.float32)
                  + b_ref[...]).astype(o_ref.dtype)


def _mm(x, w, b, bn, out_dtype=jnp.float32):
    n, k = x.shape
    m = w.shape[1]
    return pl.pallas_call(
        _mm_body,
        grid=(n // bn,),
        in_specs=[pl.BlockSpec((bn, k), lambda i: (i, 0)),
                  pl.BlockSpec((k, m), lambda i: (0, 0)),
                  pl.BlockSpec((1, m), lambda i: (0, 0))],
        out_specs=pl.BlockSpec((bn, m), lambda i: (i, 0)),
        out_shape=jax.ShapeDtypeStruct((n, m), out_dtype),
        compiler_params=pltpu.CompilerParams(
            dimension_semantics=("parallel",)),
    )(x, w, b.reshape(1, m))


def _edge_body(kinv, ga_ref, gb_ref, invT_ref, w1c_ref, w2_ref, b2_ref,
               winfT_ref, binf_ref, o_ref):
    g = ga_ref[...].astype(jnp.float32) + gb_ref[...].astype(jnp.float32)
    invT = invT_ref[...]
    w1c = w1c_ref[...]
    for k in range(kinv):
        g += invT[k, :][:, None] * w1c[k, :][None, :]
    h = jax.nn.silu(g)
    h2 = jax.nn.silu(jnp.dot(h, w2_ref[...],
                             preferred_element_type=jnp.float32) + b2_ref[...])
    gate = jax.nn.sigmoid(
        jnp.sum(h2 * winfT_ref[...], axis=1, keepdims=True) + binf_ref[0, 0])
    o_ref[...] = h2 * gate


def _edge_stage(ga, gb, invT, p, kinv, be=BE):
    ep = ga.shape[0]
    w1c = jnp.zeros((8, C), jnp.float32).at[:kinv].set(p['W1'][2 * C:2 * C + kinv])
    body = functools.partial(_edge_body, kinv)
    return pl.pallas_call(
        body,
        grid=(ep // be,),
        in_specs=[pl.BlockSpec((be, C), lambda i: (i, 0)),
                  pl.BlockSpec((be, C), lambda i: (i, 0)),
                  pl.BlockSpec((8, be), lambda i: (0, i)),
                  pl.BlockSpec((8, C), lambda i: (0, 0)),
                  pl.BlockSpec((C, C), lambda i: (0, 0)),
                  pl.BlockSpec((1, C), lambda i: (0, 0)),
                  pl.BlockSpec((1, C), lambda i: (0, 0)),
                  pl.BlockSpec((1, 1), lambda i: (0, 0))],
        out_specs=pl.BlockSpec((be, C), lambda i: (i, 0)),
        out_shape=jax.ShapeDtypeStruct((ep, C), jnp.float32),
        compiler_params=pltpu.CompilerParams(
            dimension_semantics=("parallel",)),
    )(ga, gb, invT, w1c, p['W2'], p['b2'].reshape(1, C),
      p['Winf'].reshape(1, C), p['binf'].reshape(1, 1))


def _upd_body(p_ref, x_ref, u1_ref, ub1_ref, u2_ref, ub2_ref, o_ref):
    agg = p_ref[0] + p_ref[1]
    a = jax.nn.silu(agg)
    t = jax.nn.silu(jnp.dot(a, u1_ref[...],
                            preferred_element_type=jnp.float32) + ub1_ref[...])
    o_ref[...] = x_ref[...] + jnp.dot(
        t, u2_ref[...], preferred_element_type=jnp.float32) + ub2_ref[...]


def _update(parts, x, u, bn=1000):
    n = x.shape[0]
    return pl.pallas_call(
        _upd_body,
        grid=(n // bn,),
        in_specs=[pl.BlockSpec((2, bn, C), lambda i: (0, i, 0)),
                  pl.BlockSpec((bn, C), lambda i: (i, 0)),
                  pl.BlockSpec((C, C), lambda i: (0, 0)),
                  pl.BlockSpec((1, C), lambda i: (0, 0)),
                  pl.BlockSpec((C, C), lambda i: (0, 0)),
                  pl.BlockSpec((1, C), lambda i: (0, 0))],
        out_specs=pl.BlockSpec((bn, C), lambda i: (i, 0)),
        out_shape=jax.ShapeDtypeStruct((n, C), jnp.float32),
        compiler_params=pltpu.CompilerParams(
            dimension_semantics=("parallel",)),
    )(parts, x, u['U1'], u['ub1'].reshape(1, C),
      u['U2'], u['ub2'].reshape(1, C))


def _sc_gather(tables, idxs):
    nj = len(tables)
    dt = tables[0].dtype
    cw = int(tables[0].shape[1])
    epads = [int(i.shape[0]) for i in idxs]
    mesh = plsc.VectorSubcoreMesh(core_axis_name="c", subcore_axis_name="s")
    out_type = [jax.ShapeDtypeStruct((ep, cw), dt) for ep in epads]

    @functools.partial(
        pl.kernel, mesh=mesh, out_type=out_type,
        scratch_types=([pltpu.VMEM((KG,), jnp.int32)] * KFIRE
                       + [pltpu.VMEM((KFIRE * KG, cw), dt),
                          pltpu.SemaphoreType.DMA((KFIRE,)),
                          pltpu.SemaphoreType.DMA((KFIRE,)),
                          pltpu.SemaphoreType.DMA((KFIRE,))]))
    def gather_kernel(*refs):
        table_refs = refs[:nj]
        idx_refs = refs[nj:2 * nj]
        out_refs = refs[2 * nj:3 * nj]
        idx_v = refs[3 * nj:3 * nj + KFIRE]
        rows_v, sem_i, sem_g, sem_s = refs[3 * nj + KFIRE:]
        wid = lax.axis_index("s") * NC + lax.axis_index("c")
        for j in range(nj):
            t_hbm, i_hbm, o_hbm = table_refs[j], idx_refs[j], out_refs[j]
            lw = epads[j] // NW
            ng = lw // (KG * KFIRE)
            base_w = wid * lw

            @pl.loop(0, ng)
            def _(g, t_hbm=t_hbm, i_hbm=i_hbm, o_hbm=o_hbm, base_w=base_w,
                  first=(j == 0)):
                start0 = base_w + g * (KFIRE * KG)
                for b in range(KFIRE):
                    pltpu.async_copy(i_hbm.at[0, pl.ds(start0 + b * KG, KG)],
                                     idx_v[b], sem_i.at[b])
                for b in range(KFIRE):
                    pltpu.make_async_copy(i_hbm.at[0, pl.ds(0, KG)],
                                          idx_v[b], sem_i.at[b]).wait()

                    if first:
                        @pl.when(g > 0)
                        def _(b=b, o_hbm=o_hbm):
                            pltpu.make_async_copy(
                                rows_v.at[pl.ds(0, KG)],
                                o_hbm.at[pl.ds(0, KG)], sem_s.at[b]).wait()
                    else:
                        pltpu.make_async_copy(
                            rows_v.at[pl.ds(0, KG)],
                            o_hbm.at[pl.ds(0, KG)], sem_s.at[b]).wait()

                    pltpu.async_copy(t_hbm.at[idx_v[b]],
                                     rows_v.at[pl.ds(b * KG, KG)], sem_g.at[b])
                for b in range(KFIRE):
                    pltpu.make_async_copy(t_hbm.at[idx_v[b]],
                                          rows_v.at[pl.ds(b * KG, KG)],
                                          sem_g.at[b]).wait()
                    pltpu.async_copy(rows_v.at[pl.ds(b * KG, KG)],
                                     o_hbm.at[pl.ds(start0 + b * KG, KG)],
                                     sem_s.at[b])

        for b in range(KFIRE):
            pltpu.make_async_copy(rows_v.at[pl.ds(0, KG)],
                                  out_refs[-1].at[pl.ds(0, KG)],
                                  sem_s.at[b]).wait()

    return gather_kernel(*tables, *[i.reshape(1, -1) for i in idxs])


def _sc_scatter(rank_jobs, npads, chs, accs, kgs=KGS):
    nr = len(rank_jobs)
    accmax = max(accs)
    mesh = plsc.VectorSubcoreMesh(core_axis_name="c", subcore_axis_name="s")
    out_type = [jax.ShapeDtypeStruct((NC, npads[r], C), jnp.float32)
                for r in range(nr)]
    flat_in = []
    for jobs in rank_jobs:
        for m, rv in jobs:
            flat_in.extend((m, rv))
    nin = len(flat_in)

    @functools.partial(
        pl.kernel, mesh=mesh, out_type=out_type,
        scratch_types=[pltpu.VMEM((2 * kgs, C), jnp.float32),
                       pltpu.VMEM((2, kgs), jnp.int32),
                       pltpu.VMEM((kgs,), jnp.int32),
                       pltpu.VMEM((16, C), jnp.float32),
                       pltpu.VMEM_SHARED((accmax, C), jnp.float32),
                       pltpu.SemaphoreType.DMA((2,)),
                       pltpu.SemaphoreType.DMA((2,))])
    def scatter_kernel(*refs):
        in_refs = refs[:nin]
        p_refs = refs[nin:nin + nr]
        mbuf, rbuf, sidx, zeros_v, acc, sem_m, sem_r = refs[nin + nr:]
        cid = lax.axis_index("c")
        sid = lax.axis_index("s")
        wid = sid * NC + cid

        @pl.loop(0, 16)
        def _(r):
            @pl.loop(0, C, step=16)
            def _(cc):
                zeros_v[r, pl.ds(cc, 16)] = jnp.zeros((16,), jnp.float32)

        k = 0
        for r in range(nr):
            jobs = rank_jobs[r]
            job_refs = [(in_refs[k + 2 * t], in_refs[k + 2 * t + 1])
                        for t in range(len(jobs))]
            k += 2 * len(jobs)
            ch, accr = chs[r], accs[r]
            nch = npads[r] // ch
            srows = accr // NS
            erows = ch // NS
            for c in range(nch):
                base_node = c * ch

                @pl.loop(0, srows, step=16)
                def _(r0, srows=srows):
                    pltpu.sync_copy(zeros_v, acc.at[pl.ds(sid * srows + r0, 16)])
                plsc.subcore_barrier()

                for t, (m_ref, r_ref) in enumerate(job_refs):
                    epad = int(rank_jobs[r][t][0].shape[0])
                    nbw = epad // (NW * kgs)

                    def issue(i, b, m_ref=m_ref, r_ref=r_ref, nbw=nbw):
                        base = (wid * nbw + i) * kgs
                        pltpu.async_copy(m_ref.at[pl.ds(base, kgs)],
                                         mbuf.at[pl.ds(b * kgs, kgs)],
                                         sem_m.at[b])
                        pltpu.async_copy(r_ref.at[0, pl.ds(base, kgs)],
                                         rbuf.at[b], sem_r.at[b])

                    issue(0, 0)

                    @pl.loop(0, nbw)
                    def _(i, m_ref=m_ref, r_ref=r_ref, nbw=nbw,
                          base_node=base_node, ch=ch, issue=issue):
                        b = lax.rem(i, 2)
                        nb = 1 - b

                        @pl.when(i + 1 < nbw)
                        def _():
                            issue(i + 1, nb)

                        pltpu.make_async_copy(
                            m_ref.at[pl.ds(0, kgs)],
                            mbuf.at[pl.ds(b * kgs, kgs)], sem_m.at[b]).wait()
                        pltpu.make_async_copy(
                            r_ref.at[0, pl.ds(0, kgs)],
                            rbuf.at[b], sem_r.at[b]).wait()

                        @pl.loop(0, kgs, step=16)
                        def _(tt):
                            rv = rbuf[b, pl.ds(tt, 16)] - base_node
                            ok = (rv >= 0) & (rv < ch)
                            sidx[pl.ds(tt, 16)] = jnp.where(ok, rv, ch)

                        pltpu.sync_copy(mbuf.at[pl.ds(b * kgs, kgs)],
                                        acc.at[sidx], add=True)
                plsc.subcore_barrier()

                pltpu.sync_copy(
                    acc.at[pl.ds(sid * erows, erows)],
                    p_refs[r].at[cid, pl.ds(base_node + sid * erows, erows)])
                plsc.subcore_barrier()

    return scatter_kernel(*flat_in)


def _pad1(a, n, val):
    e = a.shape[0]
    if e == n:
        return a
    return jnp.concatenate(
        [a, jnp.full((n - e,) + a.shape[1:], val, a.dtype)], axis=0)


def kernel(x_0, x_1, x_2, adj_0, adj_1, inc_1, inc_2,
           inv_rr_0, inv_rr_1, inv_rrm1_1, inv_rrm1_2, params):
    p = params
    zb = jnp.zeros((C,), jnp.float32)

    bt = jnp.float32
    a1 = _mm(x_0, p['same_0']['W1'][:C], zb, 1000, bt)
    b1t = _mm(x_0, p['same_0']['W1'][C:2 * C], p['same_0']['b1'], 1000, bt)
    a3 = _mm(x_0, p['l2h_1']['W1'][:C], zb, 1000, bt)
    a2 = _mm(x_1, p['same_1']['W1'][:C], zb, 1000, bt)
    b2t = _mm(x_1, p['same_1']['W1'][C:2 * C], p['same_1']['b1'], 1000, bt)
    b3 = _mm(x_1, p['l2h_1']['W1'][C:2 * C], p['l2h_1']['b1'], 1000, bt)
    a4 = _mm(x_1, p['l2h_2']['W1'][:C], zb, 1000, bt)
    b4 = _mm(x_2, p['l2h_2']['W1'][C:2 * C], p['l2h_2']['b1'], 1000, bt)

    convs = [
        dict(send=adj_0[0], recv=adj_0[1], inv=inv_rr_0, kinv=3,
             ta=a1, tb=b1t, p=p['same_0'], rank=0),
        dict(send=adj_1[0], recv=adj_1[1], inv=inv_rr_1, kinv=6,
             ta=a2, tb=b2t, p=p['same_1'], rank=1),
        dict(send=inc_1[0], recv=inc_1[1], inv=inv_rrm1_1, kinv=3,
             ta=a3, tb=b3, p=p['l2h_1'], rank=1),
        dict(send=inc_2[0], recv=inc_2[1], inv=inv_rrm1_2, kinv=6,
             ta=a4, tb=b4, p=p['l2h_2'], rank=2),
    ]

    gt, gi = [], []
    for cv in convs:
        e = cv['send'].shape[0]
        ep = _ceil_to(e, EALIGN)
        cv['epad'] = ep
        gt.extend((cv['ta'], cv['tb']))
        gi.extend((_pad1(cv['send'], ep, 0), _pad1(cv['recv'], ep, 0)))
        cv['recv_s'] = _pad1(cv['recv'], ep, BIG).reshape(1, ep)
        invT = jnp.transpose(cv['inv'])
        invT = jnp.concatenate(
            [invT, jnp.zeros((8 - cv['kinv'], e), jnp.float32)], axis=0)
        cv['invT'] = _pad1(invT.T, ep, 0.0).T

    for j, cv in enumerate(convs):
        ga, gb = _sc_gather([cv['ta'], cv['tb']],
                            [gi[2 * j], gi[2 * j + 1]])
        cv['ga'], cv['gb'] = ga, gb

    for cv in convs:
        cv['m'] = _edge_stage(cv['ga'], cv['gb'],
                              cv['invT'], cv['p'], cv['kinv'])

    sizes = [N0, N1, N2]
    feats = [x_0, x_1, x_2]
    outs = []
    for r in range(3):
        jobs = [(cv['m'], cv['recv_s']) for cv in convs if cv['rank'] == r]
        if r == 1:
            nch, kgs = 3, 64
        else:
            nch, kgs = max(1, -(-sizes[r] * 4 * C // (5 << 20))), KGS
        ch = _ceil_to(-(-sizes[r] // nch), NS * 8)
        acc = _ceil_to(ch + 1, 256)
        parts = _sc_scatter([jobs], [nch * ch], [ch], [acc], kgs)[0]
        u = p['upd_%d' % r]
        outs.append(_update(parts, feats[r], u, 1000))
    return tuple(outs)


# speedup vs baseline: 1.7643x; 1.0080x over previous
import functools

import jax
import jax.numpy as jnp
from jax import lax
from jax.experimental import pallas as pl
from jax.experimental.pallas import tpu as pltpu
from jax.experimental.pallas import tpu_sc as plsc

C = 128
N0, N1, N2 = 10000, 40000, 20000
NC, NS = 2, 16
NW = NC * NS
KG = 256
KFIRE = 2
KGS = 128
EALIGN = 16384
BE = 2048
BIG = 1 << 30


def _ceil_to(x, m):
    return -(-x // m) * m


def _mm_body(x_ref, w_ref, b_ref, o_ref):
    o_ref[...] = (jnp.dot(x_ref[...], w_ref[...],
                          preferred_element_type=jnp.float32)
                  + b_ref[...]).astype(o_ref.dtype)


def _mm(x, w, b, bn, out_dtype=jnp.float32):
    n, k = x.shape
    m = w.shape[1]
    return pl.pallas_call(
        _mm_body,
        grid=(n // bn,),
        in_specs=[pl.BlockSpec((bn, k), lambda i: (i, 0)),
                  pl.BlockSpec((k, m), lambda i: (0, 0)),
                  pl.BlockSpec((1, m), lambda i: (0, 0))],
        out_specs=pl.BlockSpec((bn, m), lambda i: (i, 0)),
        out_shape=jax.ShapeDtypeStruct((n, m), out_dtype),
        compiler_params=pltpu.CompilerParams(
            dimension_semantics=("parallel",)),
    )(x, w, b.reshape(1, m))


def _edge_body(kinv, ga_ref, gb_ref, invT_ref, w1c_ref, w2_ref, b2_ref,
               winfT_ref, binf_ref, o_ref):
    g = ga_ref[...].astype(jnp.float32) + gb_ref[...].astype(jnp.float32)
    invT = invT_ref[...]
    w1c = w1c_ref[...]
    for k in range(kinv):
        g += invT[k, :][:, None] * w1c[k, :][None, :]
    h = jax.nn.silu(g)
    h2 = jax.nn.silu(jnp.dot(h, w2_ref[...],
                             preferred_element_type=jnp.float32) + b2_ref[...])
    gate = jax.nn.sigmoid(
        jnp.sum(h2 * winfT_ref[...], axis=1, keepdims=True) + binf_ref[0, 0])
    o_ref[...] = h2 * gate


def _edge_stage(ga, gb, invT, p, kinv, be=BE):
    ep = ga.shape[0]
    w1c = jnp.zeros((8, C), jnp.float32).at[:kinv].set(p['W1'][2 * C:2 * C + kinv])
    body = functools.partial(_edge_body, kinv)
    return pl.pallas_call(
        body,
        grid=(ep // be,),
        in_specs=[pl.BlockSpec((be, C), lambda i: (i, 0)),
                  pl.BlockSpec((be, C), lambda i: (i, 0)),
                  pl.BlockSpec((8, be), lambda i: (0, i)),
                  pl.BlockSpec((8, C), lambda i: (0, 0)),
                  pl.BlockSpec((C, C), lambda i: (0, 0)),
                  pl.BlockSpec((1, C), lambda i: (0, 0)),
                  pl.BlockSpec((1, C), lambda i: (0, 0)),
                  pl.BlockSpec((1, 1), lambda i: (0, 0))],
        out_specs=pl.BlockSpec((be, C), lambda i: (i, 0)),
        out_shape=jax.ShapeDtypeStruct((ep, C), jnp.float32),
        compiler_params=pltpu.CompilerParams(
            dimension_semantics=("parallel",)),
    )(ga, gb, invT, w1c, p['W2'], p['b2'].reshape(1, C),
      p['Winf'].reshape(1, C), p['binf'].reshape(1, 1))


def _upd_body(p_ref, x_ref, u1_ref, ub1_ref, u2_ref, ub2_ref, o_ref):
    agg = p_ref[0] + p_ref[1]
    a = jax.nn.silu(agg)
    t = jax.nn.silu(jnp.dot(a, u1_ref[...],
                            preferred_element_type=jnp.float32) + ub1_ref[...])
    o_ref[...] = x_ref[...] + jnp.dot(
        t, u2_ref[...], preferred_element_type=jnp.float32) + ub2_ref[...]


def _update(parts, x, u, bn=2000):
    n = x.shape[0]
    return pl.pallas_call(
        _upd_body,
        grid=(n // bn,),
        in_specs=[pl.BlockSpec((2, bn, C), lambda i: (0, i, 0)),
                  pl.BlockSpec((bn, C), lambda i: (i, 0)),
                  pl.BlockSpec((C, C), lambda i: (0, 0)),
                  pl.BlockSpec((1, C), lambda i: (0, 0)),
                  pl.BlockSpec((C, C), lambda i: (0, 0)),
                  pl.BlockSpec((1, C), lambda i: (0, 0))],
        out_specs=pl.BlockSpec((bn, C), lambda i: (i, 0)),
        out_shape=jax.ShapeDtypeStruct((n, C), jnp.float32),
        compiler_params=pltpu.CompilerParams(
            dimension_semantics=("parallel",)),
    )(parts, x, u['U1'], u['ub1'].reshape(1, C),
      u['U2'], u['ub2'].reshape(1, C))


def _sc_gather(tables, idxs):
    nj = len(tables)
    dt = tables[0].dtype
    cw = int(tables[0].shape[1])
    epads = [int(i.shape[0]) for i in idxs]
    mesh = plsc.VectorSubcoreMesh(core_axis_name="c", subcore_axis_name="s")
    out_type = [jax.ShapeDtypeStruct((ep, cw), dt) for ep in epads]

    @functools.partial(
        pl.kernel, mesh=mesh, out_type=out_type,
        scratch_types=([pltpu.VMEM((KG,), jnp.int32)] * KFIRE
                       + [pltpu.VMEM((KFIRE * KG, cw), dt),
                          pltpu.SemaphoreType.DMA((KFIRE,)),
                          pltpu.SemaphoreType.DMA((KFIRE,)),
                          pltpu.SemaphoreType.DMA((KFIRE,))]))
    def gather_kernel(*refs):
        table_refs = refs[:nj]
        idx_refs = refs[nj:2 * nj]
        out_refs = refs[2 * nj:3 * nj]
        idx_v = refs[3 * nj:3 * nj + KFIRE]
        rows_v, sem_i, sem_g, sem_s = refs[3 * nj + KFIRE:]
        wid = lax.axis_index("s") * NC + lax.axis_index("c")
        for j in range(nj):
            t_hbm, i_hbm, o_hbm = table_refs[j], idx_refs[j], out_refs[j]
            lw = epads[j] // NW
            ng = lw // (KG * KFIRE)
            base_w = wid * lw

            @pl.loop(0, ng)
            def _(g, t_hbm=t_hbm, i_hbm=i_hbm, o_hbm=o_hbm, base_w=base_w,
                  first=(j == 0)):
                start0 = base_w + g * (KFIRE * KG)
                for b in range(KFIRE):
                    pltpu.async_copy(i_hbm.at[0, pl.ds(start0 + b * KG, KG)],
                                     idx_v[b], sem_i.at[b])
                for b in range(KFIRE):
                    pltpu.make_async_copy(i_hbm.at[0, pl.ds(0, KG)],
                                          idx_v[b], sem_i.at[b]).wait()

                    if first:
                        @pl.when(g > 0)
                        def _(b=b, o_hbm=o_hbm):
                            pltpu.make_async_copy(
                                rows_v.at[pl.ds(0, KG)],
                                o_hbm.at[pl.ds(0, KG)], sem_s.at[b]).wait()
                    else:
                        pltpu.make_async_copy(
                            rows_v.at[pl.ds(0, KG)],
                            o_hbm.at[pl.ds(0, KG)], sem_s.at[b]).wait()

                    pltpu.async_copy(t_hbm.at[idx_v[b]],
                                     rows_v.at[pl.ds(b * KG, KG)], sem_g.at[b])
                for b in range(KFIRE):
                    pltpu.make_async_copy(t_hbm.at[idx_v[b]],
                                          rows_v.at[pl.ds(b * KG, KG)],
                                          sem_g.at[b]).wait()
                    pltpu.async_copy(rows_v.at[pl.ds(b * KG, KG)],
                                     o_hbm.at[pl.ds(start0 + b * KG, KG)],
                                     sem_s.at[b])

        for b in range(KFIRE):
            pltpu.make_async_copy(rows_v.at[pl.ds(0, KG)],
                                  out_refs[-1].at[pl.ds(0, KG)],
                                  sem_s.at[b]).wait()

    return gather_kernel(*tables, *[i.reshape(1, -1) for i in idxs])


def _sc_scatter(rank_jobs, npads, chs, accs, kgs=KGS):
    nr = len(rank_jobs)
    accmax = max(accs)
    mesh = plsc.VectorSubcoreMesh(core_axis_name="c", subcore_axis_name="s")
    out_type = [jax.ShapeDtypeStruct((NC, npads[r], C), jnp.float32)
                for r in range(nr)]
    flat_in = []
    for jobs in rank_jobs:
        for m, rv in jobs:
            flat_in.extend((m, rv))
    nin = len(flat_in)

    @functools.partial(
        pl.kernel, mesh=mesh, out_type=out_type,
        scratch_types=[pltpu.VMEM((2 * kgs, C), jnp.float32),
                       pltpu.VMEM((2, kgs), jnp.int32),
                       pltpu.VMEM((kgs,), jnp.int32),
                       pltpu.VMEM((16, C), jnp.float32),
                       pltpu.VMEM_SHARED((accmax, C), jnp.float32),
                       pltpu.SemaphoreType.DMA((2,)),
                       pltpu.SemaphoreType.DMA((2,))])
    def scatter_kernel(*refs):
        in_refs = refs[:nin]
        p_refs = refs[nin:nin + nr]
        mbuf, rbuf, sidx, zeros_v, acc, sem_m, sem_r = refs[nin + nr:]
        cid = lax.axis_index("c")
        sid = lax.axis_index("s")
        wid = sid * NC + cid

        @pl.loop(0, 16)
        def _(r):
            @pl.loop(0, C, step=16)
            def _(cc):
                zeros_v[r, pl.ds(cc, 16)] = jnp.zeros((16,), jnp.float32)

        k = 0
        for r in range(nr):
            jobs = rank_jobs[r]
            job_refs = [(in_refs[k + 2 * t], in_refs[k + 2 * t + 1])
                        for t in range(len(jobs))]
            k += 2 * len(jobs)
            ch, accr = chs[r], accs[r]
            nch = npads[r] // ch
            srows = accr // NS
            erows = ch // NS
            for c in range(nch):
                base_node = c * ch

                @pl.loop(0, srows, step=16)
                def _(r0, srows=srows):
                    pltpu.sync_copy(zeros_v, acc.at[pl.ds(sid * srows + r0, 16)])
                plsc.subcore_barrier()

                for t, (m_ref, r_ref) in enumerate(job_refs):
                    epad = int(rank_jobs[r][t][0].shape[0])
                    nbw = epad // (NW * kgs)

                    def issue(i, b, m_ref=m_ref, r_ref=r_ref, nbw=nbw):
                        base = (wid * nbw + i) * kgs
                        pltpu.async_copy(m_ref.at[pl.ds(base, kgs)],
                                         mbuf.at[pl.ds(b * kgs, kgs)],
                                         sem_m.at[b])
                        pltpu.async_copy(r_ref.at[0, pl.ds(base, kgs)],
                                         rbuf.at[b], sem_r.at[b])

                    issue(0, 0)

                    @pl.loop(0, nbw)
                    def _(i, m_ref=m_ref, r_ref=r_ref, nbw=nbw,
                          base_node=base_node, ch=ch, issue=issue):
                        b = lax.rem(i, 2)
                        nb = 1 - b

                        @pl.when(i + 1 < nbw)
                        def _():
                            issue(i + 1, nb)

                        pltpu.make_async_copy(
                            m_ref.at[pl.ds(0, kgs)],
                            mbuf.at[pl.ds(b * kgs, kgs)], sem_m.at[b]).wait()
                        pltpu.make_async_copy(
                            r_ref.at[0, pl.ds(0, kgs)],
                            rbuf.at[b], sem_r.at[b]).wait()

                        @pl.loop(0, kgs, step=16)
                        def _(tt):
                            rv = rbuf[b, pl.ds(tt, 16)] - base_node
                            ok = (rv >= 0) & (rv < ch)
                            sidx[pl.ds(tt, 16)] = jnp.where(ok, rv, ch)

                        pltpu.sync_copy(mbuf.at[pl.ds(b * kgs, kgs)],
                                        acc.at[sidx], add=True)
                plsc.subcore_barrier()

                pltpu.sync_copy(
                    acc.at[pl.ds(sid * erows, erows)],
                    p_refs[r].at[cid, pl.ds(base_node + sid * erows, erows)])
                plsc.subcore_barrier()

    return scatter_kernel(*flat_in)


def _pad1(a, n, val):
    e = a.shape[0]
    if e == n:
        return a
    return jnp.concatenate(
        [a, jnp.full((n - e,) + a.shape[1:], val, a.dtype)], axis=0)


def kernel(x_0, x_1, x_2, adj_0, adj_1, inc_1, inc_2,
           inv_rr_0, inv_rr_1, inv_rrm1_1, inv_rrm1_2, params):
    p = params
    zb = jnp.zeros((C,), jnp.float32)

    bt = jnp.float32
    a1 = _mm(x_0, p['same_0']['W1'][:C], zb, 2000, bt)
    b1t = _mm(x_0, p['same_0']['W1'][C:2 * C], p['same_0']['b1'], 2000, bt)
    a3 = _mm(x_0, p['l2h_1']['W1'][:C], zb, 2000, bt)
    a2 = _mm(x_1, p['same_1']['W1'][:C], zb, 2000, bt)
    b2t = _mm(x_1, p['same_1']['W1'][C:2 * C], p['same_1']['b1'], 2000, bt)
    b3 = _mm(x_1, p['l2h_1']['W1'][C:2 * C], p['l2h_1']['b1'], 2000, bt)
    a4 = _mm(x_1, p['l2h_2']['W1'][:C], zb, 2000, bt)
    b4 = _mm(x_2, p['l2h_2']['W1'][C:2 * C], p['l2h_2']['b1'], 2000, bt)

    convs = [
        dict(send=adj_0[0], recv=adj_0[1], inv=inv_rr_0, kinv=3,
             ta=a1, tb=b1t, p=p['same_0'], rank=0),
        dict(send=adj_1[0], recv=adj_1[1], inv=inv_rr_1, kinv=6,
             ta=a2, tb=b2t, p=p['same_1'], rank=1),
        dict(send=inc_1[0], recv=inc_1[1], inv=inv_rrm1_1, kinv=3,
             ta=a3, tb=b3, p=p['l2h_1'], rank=1),
        dict(send=inc_2[0], recv=inc_2[1], inv=inv_rrm1_2, kinv=6,
             ta=a4, tb=b4, p=p['l2h_2'], rank=2),
    ]

    gt, gi = [], []
    for cv in convs:
        e = cv['send'].shape[0]
        ep = _ceil_to(e, EALIGN)
        cv['epad'] = ep
        gt.extend((cv['ta'], cv['tb']))
        gi.extend((_pad1(cv['send'], ep, 0), _pad1(cv['recv'], ep, 0)))
        cv['recv_s'] = _pad1(cv['recv'], ep, BIG).reshape(1, ep)
        invT = jnp.transpose(cv['inv'])
        invT = jnp.concatenate(
            [invT, jnp.zeros((8 - cv['kinv'], e), jnp.float32)], axis=0)
        cv['invT'] = _pad1(invT.T, ep, 0.0).T

    for j, cv in enumerate(convs):
        ga, gb = _sc_gather([cv['ta'], cv['tb']],
                            [gi[2 * j], gi[2 * j + 1]])
        cv['ga'], cv['gb'] = ga, gb

    for cv in convs:
        cv['m'] = _edge_stage(cv['ga'], cv['gb'],
                              cv['invT'], cv['p'], cv['kinv'])

    sizes = [N0, N1, N2]
    feats = [x_0, x_1, x_2]
    outs = []
    for r in range(3):
        jobs = [(cv['m'], cv['recv_s']) for cv in convs if cv['rank'] == r]
        if r == 1:
            nch, kgs = 3, 64
        else:
            nch, kgs = max(1, -(-sizes[r] * 4 * C // (5 << 20))), KGS
        ch = _ceil_to(-(-sizes[r] // nch), NS * 8)
        acc = _ceil_to(ch + 1, 256)
        parts = _sc_scatter([jobs], [nch * ch], [ch], [acc], kgs)[0]
        u = p['upd_%d' % r]
        outs.append(_update(parts, feats[r], u, 2000))
    return tuple(outs)
